# Initial kernel scaffold; baseline (speedup 1.0000x reference)
import jax
import jax.numpy as jnp
from jax.experimental import pallas as pl


def kernel(z, edge_index, W0, b0, W1, b1, Wl, bl):
    raise NotImplementedError("write your pallas kernel here")


# speedup vs baseline: 10.1994x; 10.1994x over previous
import functools

import jax
import jax.numpy as jnp
from jax import lax
from jax.experimental import pallas as pl
from jax.experimental.pallas import tpu as pltpu
from jax.experimental.pallas import tpu_sc as plsc

N = 10000
NP = 10240
E = 320000
EP = 321536
IN_C = 128
HID = 256
OUT_C = 128
HALF = HID // 2

NCORE = 2
NSUB = 16
ROWS_PER_TILE = NP // NSUB

CHUNK = 128
CPT = EP // (NSUB * CHUNK)
DEG_CHUNK = 64
DEG_CPT = EP // (NSUB * NCORE * DEG_CHUNK)
DEG_W = 16

_mesh = plsc.VectorSubcoreMesh(core_axis_name="c", subcore_axis_name="s")


@functools.partial(
    pl.kernel,
    out_type=jax.ShapeDtypeStruct((NCORE, NP, DEG_W), jnp.float32),
    mesh=_mesh,
    scratch_types=[
        pltpu.VMEM((DEG_CHUNK,), jnp.int32),
        pltpu.VMEM((DEG_CHUNK, DEG_W), jnp.float32),
        pltpu.VMEM_SHARED((NP, DEG_W), jnp.float32),
    ],
)
def _deg_kernel(dst_hbm, zeros_hbm, deg_out, idx_v, ones_v, acc_sh):
    c = lax.axis_index("c")
    s = lax.axis_index("s")
    pltpu.sync_copy(zeros_hbm, acc_sh.at[pl.ds(s * ROWS_PER_TILE, ROWS_PER_TILE)])
    for i in range(DEG_CHUNK):
        ones_v[i, :] = jnp.full((DEG_W,), 1.0, jnp.float32)
    plsc.subcore_barrier()
    wid = s * NCORE + c

    def body(k, carry):
        off = (wid * DEG_CPT + k) * DEG_CHUNK
        pltpu.sync_copy(dst_hbm.at[pl.ds(off, DEG_CHUNK)], idx_v)
        pltpu.sync_copy(ones_v, acc_sh.at[idx_v], add=True)
        return carry

    lax.fori_loop(0, DEG_CPT, body, 0)
    plsc.subcore_barrier()
    pltpu.sync_copy(
        acc_sh.at[pl.ds(s * ROWS_PER_TILE, ROWS_PER_TILE)],
        deg_out.at[c, pl.ds(s * ROWS_PER_TILE, ROWS_PER_TILE)],
    )


@functools.partial(
    pl.kernel,
    out_type=jax.ShapeDtypeStruct((NCORE, NP, HALF), jnp.float32),
    mesh=_mesh,
    scratch_types=[
        pltpu.VMEM((CHUNK,), jnp.int32),
        pltpu.VMEM((CHUNK,), jnp.int32),
        pltpu.VMEM((CHUNK, HALF), jnp.float32),
        pltpu.VMEM_SHARED((NP, HALF), jnp.float32),
        pltpu.SemaphoreType.DMA,
    ],
)
def _agg_kernel(hs_hbm, src_hbm, dst_hbm, zeros_hbm, out_hbm,
                src_v, dst_v, rows_v, acc_sh, sem):
    c = lax.axis_index("c")
    s = lax.axis_index("s")
    pltpu.sync_copy(zeros_hbm, acc_sh.at[pl.ds(s * ROWS_PER_TILE, ROWS_PER_TILE)])
    plsc.subcore_barrier()
    table = hs_hbm.at[c]

    def body(k, carry):
        off = (s * CPT + k) * CHUNK
        pltpu.sync_copy(src_hbm.at[pl.ds(off, CHUNK)], src_v)
        pltpu.sync_copy(dst_hbm.at[pl.ds(off, CHUNK)], dst_v)
        pltpu.async_copy(table.at[src_v], rows_v, sem).wait()
        pltpu.sync_copy(rows_v, acc_sh.at[dst_v], add=True)
        return carry

    lax.fori_loop(0, CPT, body, 0)
    plsc.subcore_barrier()
    pltpu.sync_copy(
        acc_sh.at[pl.ds(s * ROWS_PER_TILE, ROWS_PER_TILE)],
        out_hbm.at[c, pl.ds(s * ROWS_PER_TILE, ROWS_PER_TILE)],
    )


RB = 1024


def _dis_from_deg(deg_blk):
    d = deg_blk[0, :, 0] + deg_blk[1, :, 0] + 1.0
    return (1.0 / jnp.sqrt(d))[:, None]


def _tc1_body(z_ref, w_ref, deg_ref, hs_ref):
    dis = _dis_from_deg(deg_ref[...])
    h = jnp.dot(z_ref[...], w_ref[...], preferred_element_type=jnp.float32)
    hs = dis * h
    hs_ref[0] = hs[:, :HALF]
    hs_ref[1] = hs[:, HALF:]


def _tc_mid_body(agg_ref, hsp_ref, deg_ref, w_ref, b_ref, out_ref):
    dis = _dis_from_deg(deg_ref[...])
    pre = jnp.concatenate(
        [agg_ref[0] + hsp_ref[0], agg_ref[1] + hsp_ref[1]], axis=1)
    t = jnp.maximum(dis * pre + b_ref[0], 0.0)
    h = jnp.dot(t, w_ref[...], preferred_element_type=jnp.float32)
    hs = dis * h
    out_ref[0] = hs[:, :HALF]
    out_ref[1] = hs[:, HALF:]


def _tc_out_body(agg_ref, hsp_ref, deg_ref, w_ref, b_ref, bl_ref, out_ref):
    dis = _dis_from_deg(deg_ref[...])
    pre = jnp.concatenate(
        [agg_ref[0] + hsp_ref[0], agg_ref[1] + hsp_ref[1]], axis=1)
    t = jnp.maximum(dis * pre + b_ref[0], 0.0)
    h = jnp.dot(t, w_ref[...], preferred_element_type=jnp.float32)
    out_ref[...] = jax.nn.sigmoid(h + bl_ref[0])


def _stacked_spec():
    return pl.BlockSpec((2, RB, HALF), lambda i: (0, i, 0))


def _tc1(z_p, W0, deg2):
    return pl.pallas_call(
        _tc1_body,
        grid=(NP // RB,),
        in_specs=[
            pl.BlockSpec((RB, IN_C), lambda i: (i, 0)),
            pl.BlockSpec((IN_C, HID), lambda i: (0, 0)),
            pl.BlockSpec((2, RB, DEG_W), lambda i: (0, i, 0)),
        ],
        out_specs=_stacked_spec(),
        out_shape=jax.ShapeDtypeStruct((2, NP, HALF), jnp.float32),
    )(z_p, W0, deg2)


def _tc_mid(agg, hsp, deg2, W1, b0):
    return pl.pallas_call(
        _tc_mid_body,
        grid=(NP // RB,),
        in_specs=[
            _stacked_spec(),
            _stacked_spec(),
            pl.BlockSpec((2, RB, DEG_W), lambda i: (0, i, 0)),
            pl.BlockSpec((HID, HID), lambda i: (0, 0)),
            pl.BlockSpec((1, HID), lambda i: (0, 0)),
        ],
        out_specs=_stacked_spec(),
        out_shape=jax.ShapeDtypeStruct((2, NP, HALF), jnp.float32),
    )(agg, hsp, deg2, W1, b0.reshape(1, HID))


def _tc_out(agg, hsp, deg2, Wl, b1, bl):
    return pl.pallas_call(
        _tc_out_body,
        grid=(NP // RB,),
        in_specs=[
            _stacked_spec(),
            _stacked_spec(),
            pl.BlockSpec((2, RB, DEG_W), lambda i: (0, i, 0)),
            pl.BlockSpec((HID, OUT_C), lambda i: (0, 0)),
            pl.BlockSpec((1, HID), lambda i: (0, 0)),
            pl.BlockSpec((1, OUT_C), lambda i: (0, 0)),
        ],
        out_specs=pl.BlockSpec((RB, OUT_C), lambda i: (i, 0)),
        out_shape=jax.ShapeDtypeStruct((NP, OUT_C), jnp.float32),
    )(agg, hsp, deg2, Wl, b1.reshape(1, HID), bl.reshape(1, OUT_C))


def kernel(z, edge_index, W0, b0, W1, b1, Wl, bl):
    src = edge_index[0]
    dst = edge_index[1]
    pad = (N + (jnp.arange(EP - E) % (NP - N))).astype(jnp.int32)
    src_p = jnp.concatenate([src, pad])
    dst_p = jnp.concatenate([dst, pad])
    z_p = jnp.zeros((NP, IN_C), z.dtype).at[:N].set(z)

    zeros_deg = jnp.zeros((ROWS_PER_TILE, DEG_W), jnp.float32)
    zeros_agg = jnp.zeros((ROWS_PER_TILE, HALF), jnp.float32)

    deg2 = _deg_kernel(dst_p, zeros_deg)
    hs0 = _tc1(z_p, W0, deg2)
    agg0 = _agg_kernel(hs0, src_p, dst_p, zeros_agg)
    hs1 = _tc_mid(agg0, hs0, deg2, W1, b0)
    agg1 = _agg_kernel(hs1, src_p, dst_p, zeros_agg)
    out = _tc_out(agg1, hs1, deg2, Wl, b1, bl)
    return out[:N]


# speedup vs baseline: 14.2933x; 1.4014x over previous
import functools

import jax
import jax.numpy as jnp
from jax import lax
from jax.experimental import pallas as pl
from jax.experimental.pallas import tpu as pltpu
from jax.experimental.pallas import tpu_sc as plsc

N = 10000
NP = 10240
E = 320000
EP = 321536
IN_C = 128
HID = 256
OUT_C = 128
HALF = HID // 2

NCORE = 2
NSUB = 16
ROWS_PER_TILE = NP // NSUB

CHUNK = 128
CPT = EP // (NSUB * CHUNK)
NCHUNK = NSUB * CPT + 1
EP2 = NCHUNK * CHUNK
DEG_CHUNK = 64
DEG_CPT = EP // (NSUB * NCORE * DEG_CHUNK)
DEG_W = 16

_mesh = plsc.VectorSubcoreMesh(core_axis_name="c", subcore_axis_name="s")


@functools.partial(
    pl.kernel,
    out_type=jax.ShapeDtypeStruct((NCORE, NP, DEG_W), jnp.float32),
    mesh=_mesh,
    scratch_types=[
        pltpu.VMEM((DEG_CHUNK,), jnp.int32),
        pltpu.VMEM((DEG_CHUNK, DEG_W), jnp.float32),
        pltpu.VMEM_SHARED((NP, DEG_W), jnp.float32),
    ],
)
def _deg_kernel(dst_hbm, zeros_hbm, deg_out, idx_v, ones_v, acc_sh):
    c = lax.axis_index("c")
    s = lax.axis_index("s")
    pltpu.sync_copy(zeros_hbm, acc_sh.at[pl.ds(s * ROWS_PER_TILE, ROWS_PER_TILE)])
    for i in range(DEG_CHUNK):
        ones_v[i, :] = jnp.full((DEG_W,), 1.0, jnp.float32)
    plsc.subcore_barrier()
    wid = s * NCORE + c

    def body(k, carry):
        off = (wid * DEG_CPT + k) * DEG_CHUNK
        pltpu.sync_copy(dst_hbm.at[pl.ds(off, DEG_CHUNK)], idx_v)
        pltpu.sync_copy(ones_v, acc_sh.at[idx_v], add=True)
        return carry

    lax.fori_loop(0, DEG_CPT, body, 0)
    plsc.subcore_barrier()
    pltpu.sync_copy(
        acc_sh.at[pl.ds(s * ROWS_PER_TILE, ROWS_PER_TILE)],
        deg_out.at[c, pl.ds(s * ROWS_PER_TILE, ROWS_PER_TILE)],
    )


@functools.partial(
    pl.kernel,
    out_type=jax.ShapeDtypeStruct((NCORE, NP, HALF), jnp.float32),
    mesh=_mesh,
    scratch_types=[
        pltpu.VMEM((2, CHUNK), jnp.int32),
        pltpu.VMEM((2, CHUNK), jnp.int32),
        pltpu.VMEM((CHUNK, HALF), jnp.float32),
        pltpu.VMEM((CHUNK, HALF), jnp.float32),
        pltpu.VMEM_SHARED((NP, HALF), jnp.float32),
        pltpu.SemaphoreType.DMA,
        pltpu.SemaphoreType.DMA,
        pltpu.SemaphoreType.DMA,
        pltpu.SemaphoreType.DMA,
    ],
)
def _agg_kernel(hs_hbm, ei_hbm, zeros_hbm, out_hbm,
                idx0, idx1, rows0, rows1, acc_sh, g0, g1, s0, s1):
    c = lax.axis_index("c")
    s = lax.axis_index("s")
    pltpu.sync_copy(zeros_hbm, acc_sh.at[pl.ds(s * ROWS_PER_TILE, ROWS_PER_TILE)])
    plsc.subcore_barrier()
    table = hs_hbm.at[c]
    idx = (idx0, idx1)
    rows = (rows0, rows1)
    gsem = (g0, g1)
    ssem = (s0, s1)

    def idx_load(cc, b):
        pltpu.sync_copy(ei_hbm.at[s * CPT + cc], idx[b])

    def gather_start(b):
        pltpu.async_copy(table.at[idx[b].at[0]], rows[b], gsem[b])

    def gather_wait(b):
        pltpu.make_async_copy(table.at[idx[b].at[0]], rows[b], gsem[b]).wait()

    def scat_start(b):
        pltpu.async_copy(rows[b], acc_sh.at[idx[b].at[1]], ssem[b], add=True)

    def scat_wait(b):
        pltpu.make_async_copy(rows[b], acc_sh.at[idx[b].at[1]], ssem[b]).wait()

    def half(b, cc):
        gather_wait(b)
        scat_start(b)
        scat_wait(1 - b)
        idx_load(cc + 1, 1 - b)
        gather_start(1 - b)

    idx_load(0, 0)
    gather_start(0)
    gather_wait(0)
    scat_start(0)
    idx_load(1, 1)
    gather_start(1)

    def body(j, carry):
        half(1, 2 * j + 1)
        half(0, 2 * j + 2)
        return carry

    lax.fori_loop(0, (CPT - 1) // 2, body, 0)
    gather_wait(1)
    scat_wait(0)
    plsc.subcore_barrier()
    pltpu.sync_copy(
        acc_sh.at[pl.ds(s * ROWS_PER_TILE, ROWS_PER_TILE)],
        out_hbm.at[c, pl.ds(s * ROWS_PER_TILE, ROWS_PER_TILE)],
    )


RB = 1024


def _dis_from_deg(deg_blk):
    d = deg_blk[0, :, 0] + deg_blk[1, :, 0] + 1.0
    return (1.0 / jnp.sqrt(d))[:, None]


def _tc1_body(z_ref, w_ref, deg_ref, hs_ref):
    dis = _dis_from_deg(deg_ref[...])
    h = jnp.dot(z_ref[...], w_ref[...], preferred_element_type=jnp.float32)
    hs = dis * h
    hs_ref[0] = hs[:, :HALF]
    hs_ref[1] = hs[:, HALF:]


def _tc_mid_body(agg_ref, hsp_ref, deg_ref, w_ref, b_ref, out_ref):
    dis = _dis_from_deg(deg_ref[...])
    pre = jnp.concatenate(
        [agg_ref[0] + hsp_ref[0], agg_ref[1] + hsp_ref[1]], axis=1)
    t = jnp.maximum(dis * pre + b_ref[0], 0.0)
    h = jnp.dot(t, w_ref[...], preferred_element_type=jnp.float32)
    hs = dis * h
    out_ref[0] = hs[:, :HALF]
    out_ref[1] = hs[:, HALF:]


def _tc_out_body(agg_ref, hsp_ref, deg_ref, w_ref, b_ref, bl_ref, out_ref):
    dis = _dis_from_deg(deg_ref[...])
    pre = jnp.concatenate(
        [agg_ref[0] + hsp_ref[0], agg_ref[1] + hsp_ref[1]], axis=1)
    t = jnp.maximum(dis * pre + b_ref[0], 0.0)
    h = jnp.dot(t, w_ref[...], preferred_element_type=jnp.float32)
    out_ref[...] = jax.nn.sigmoid(h + bl_ref[0])


def _stacked_spec():
    return pl.BlockSpec((2, RB, HALF), lambda i: (0, i, 0))


def _tc1(z_p, W0, deg2):
    return pl.pallas_call(
        _tc1_body,
        grid=(NP // RB,),
        in_specs=[
            pl.BlockSpec((RB, IN_C), lambda i: (i, 0)),
            pl.BlockSpec((IN_C, HID), lambda i: (0, 0)),
            pl.BlockSpec((2, RB, DEG_W), lambda i: (0, i, 0)),
        ],
        out_specs=_stacked_spec(),
        out_shape=jax.ShapeDtypeStruct((2, NP, HALF), jnp.float32),
    )(z_p, W0, deg2)


def _tc_mid(agg, hsp, deg2, W1, b0):
    return pl.pallas_call(
        _tc_mid_body,
        grid=(NP // RB,),
        in_specs=[
            _stacked_spec(),
            _stacked_spec(),
            pl.BlockSpec((2, RB, DEG_W), lambda i: (0, i, 0)),
            pl.BlockSpec((HID, HID), lambda i: (0, 0)),
            pl.BlockSpec((1, HID), lambda i: (0, 0)),
        ],
        out_specs=_stacked_spec(),
        out_shape=jax.ShapeDtypeStruct((2, NP, HALF), jnp.float32),
    )(agg, hsp, deg2, W1, b0.reshape(1, HID))


def _tc_out(agg, hsp, deg2, Wl, b1, bl):
    return pl.pallas_call(
        _tc_out_body,
        grid=(NP // RB,),
        in_specs=[
            _stacked_spec(),
            _stacked_spec(),
            pl.BlockSpec((2, RB, DEG_W), lambda i: (0, i, 0)),
            pl.BlockSpec((HID, OUT_C), lambda i: (0, 0)),
            pl.BlockSpec((1, HID), lambda i: (0, 0)),
            pl.BlockSpec((1, OUT_C), lambda i: (0, 0)),
        ],
        out_specs=pl.BlockSpec((RB, OUT_C), lambda i: (i, 0)),
        out_shape=jax.ShapeDtypeStruct((NP, OUT_C), jnp.float32),
    )(agg, hsp, deg2, Wl, b1.reshape(1, HID), bl.reshape(1, OUT_C))


def kernel(z, edge_index, W0, b0, W1, b1, Wl, bl):
    src = edge_index[0]
    dst = edge_index[1]
    pad = (N + (jnp.arange(EP2 - E) % (NP - N))).astype(jnp.int32)
    src_p = jnp.concatenate([src, pad])
    dst_p = jnp.concatenate([dst, pad])
    ei_chunks = jnp.stack(
        [src_p.reshape(NCHUNK, CHUNK), dst_p.reshape(NCHUNK, CHUNK)], axis=1)
    z_p = jnp.zeros((NP, IN_C), z.dtype).at[:N].set(z)

    zeros_deg = jnp.zeros((ROWS_PER_TILE, DEG_W), jnp.float32)
    zeros_agg = jnp.zeros((ROWS_PER_TILE, HALF), jnp.float32)

    deg2 = _deg_kernel(dst_p[:EP], zeros_deg)
    hs0 = _tc1(z_p, W0, deg2)
    agg0 = _agg_kernel(hs0, ei_chunks, zeros_agg)
    hs1 = _tc_mid(agg0, hs0, deg2, W1, b0)
    agg1 = _agg_kernel(hs1, ei_chunks, zeros_agg)
    out = _tc_out(agg1, hs1, deg2, Wl, b1, bl)
    return out[:N]


# speedup vs baseline: 15.5205x; 1.0859x over previous
import functools

import jax
import jax.numpy as jnp
from jax import lax
from jax.experimental import pallas as pl
from jax.experimental.pallas import tpu as pltpu
from jax.experimental.pallas import tpu_sc as plsc

N = 10000
NP = 10240
E = 320000
IN_C = 128
HID = 256
OUT_C = 128
HALF = HID // 2

NCORE = 2
NSUB = 16
ROWS_PER_TILE = NP // NSUB

CHUNK = 128
CPT = 159
NCHUNK = NSUB * CPT + 1
EP2 = NCHUNK * CHUNK
DEG_CHUNK = 64
DEG_CPT = 160
EP_DEG = NCORE * NSUB * DEG_CPT * DEG_CHUNK
DEG_W = 16

_mesh = plsc.VectorSubcoreMesh(core_axis_name="c", subcore_axis_name="s")


DEG_DEPTH = 8


@functools.partial(
    pl.kernel,
    out_type=jax.ShapeDtypeStruct((NCORE, NP, DEG_W), jnp.float32),
    mesh=_mesh,
    scratch_types=[
        pltpu.VMEM((DEG_CPT, DEG_CHUNK), jnp.int32),
        pltpu.VMEM((DEG_CHUNK, DEG_W), jnp.float32),
        pltpu.VMEM_SHARED((NP, DEG_W), jnp.float32),
        pltpu.SemaphoreType.DMA,
    ],
)
def _deg_kernel(dst_hbm, zeros_hbm, deg_out, idx_all, ones_v, acc_sh, sem):
    c = lax.axis_index("c")
    s = lax.axis_index("s")
    pltpu.sync_copy(zeros_hbm, acc_sh.at[pl.ds(s * ROWS_PER_TILE, ROWS_PER_TILE)])
    wid = s * NCORE + c
    pltpu.sync_copy(dst_hbm.at[pl.ds(wid * DEG_CPT, DEG_CPT)], idx_all)
    for i in range(DEG_CHUNK):
        ones_v[i, :] = jnp.full((DEG_W,), 1.0, jnp.float32)
    plsc.subcore_barrier()

    def wait_one():
        pltpu.make_async_copy(ones_v, acc_sh.at[idx_all.at[0]], sem).wait()

    def body(k, carry):
        pltpu.async_copy(ones_v, acc_sh.at[idx_all.at[k]], sem, add=True)

        @pl.when(k >= DEG_DEPTH)
        def _():
            wait_one()

        return carry

    lax.fori_loop(0, DEG_CPT, body, 0)
    for _ in range(DEG_DEPTH):
        wait_one()
    plsc.subcore_barrier()
    pltpu.sync_copy(
        acc_sh.at[pl.ds(s * ROWS_PER_TILE, ROWS_PER_TILE)],
        deg_out.at[c, pl.ds(s * ROWS_PER_TILE, ROWS_PER_TILE)],
    )


@functools.partial(
    pl.kernel,
    out_type=jax.ShapeDtypeStruct((NCORE, NP, HALF), jnp.float32),
    mesh=_mesh,
    scratch_types=[
        pltpu.VMEM((2, CHUNK), jnp.int32),
        pltpu.VMEM((2, CHUNK), jnp.int32),
        pltpu.VMEM((CHUNK, HALF), jnp.float32),
        pltpu.VMEM((CHUNK, HALF), jnp.float32),
        pltpu.VMEM_SHARED((NP, HALF), jnp.float32),
        pltpu.SemaphoreType.DMA,
        pltpu.SemaphoreType.DMA,
        pltpu.SemaphoreType.DMA,
        pltpu.SemaphoreType.DMA,
    ],
)
def _agg_kernel(hs_hbm, ei_hbm, zeros_hbm, out_hbm,
                idx0, idx1, rows0, rows1, acc_sh, g0, g1, s0, s1):
    c = lax.axis_index("c")
    s = lax.axis_index("s")
    pltpu.sync_copy(zeros_hbm, acc_sh.at[pl.ds(s * ROWS_PER_TILE, ROWS_PER_TILE)])
    plsc.subcore_barrier()
    table = hs_hbm.at[c]
    idx = (idx0, idx1)
    rows = (rows0, rows1)
    gsem = (g0, g1)
    ssem = (s0, s1)

    def idx_load(cc, b):
        pltpu.sync_copy(ei_hbm.at[s * CPT + cc], idx[b])

    def gather_start(b):
        pltpu.async_copy(table.at[idx[b].at[0]], rows[b], gsem[b])

    def gather_wait(b):
        pltpu.make_async_copy(table.at[idx[b].at[0]], rows[b], gsem[b]).wait()

    def scat_start(b):
        pltpu.async_copy(rows[b], acc_sh.at[idx[b].at[1]], ssem[b], add=True)

    def scat_wait(b):
        pltpu.make_async_copy(rows[b], acc_sh.at[idx[b].at[1]], ssem[b]).wait()

    def half(b, cc):
        gather_wait(b)
        scat_start(b)
        scat_wait(1 - b)
        idx_load(cc + 1, 1 - b)
        gather_start(1 - b)

    idx_load(0, 0)
    gather_start(0)
    gather_wait(0)
    scat_start(0)
    idx_load(1, 1)
    gather_start(1)

    def body(j, carry):
        half(1, 2 * j + 1)
        half(0, 2 * j + 2)
        return carry

    lax.fori_loop(0, (CPT - 1) // 2, body, 0)
    gather_wait(1)
    scat_wait(0)
    plsc.subcore_barrier()
    pltpu.sync_copy(
        acc_sh.at[pl.ds(s * ROWS_PER_TILE, ROWS_PER_TILE)],
        out_hbm.at[c, pl.ds(s * ROWS_PER_TILE, ROWS_PER_TILE)],
    )


RB = 1024


def _dis_from_deg(deg_blk):
    d = deg_blk[0, :, 0] + deg_blk[1, :, 0] + 1.0
    return (1.0 / jnp.sqrt(d))[:, None]


def _tc1_body(z_ref, w_ref, deg_ref, hs_ref):
    dis = _dis_from_deg(deg_ref[...])
    h = jnp.dot(z_ref[...], w_ref[...], preferred_element_type=jnp.float32)
    hs = dis * h
    hs_ref[0] = hs[:, :HALF]
    hs_ref[1] = hs[:, HALF:]


def _tc_mid_body(agg_ref, hsp_ref, deg_ref, w_ref, b_ref, out_ref):
    dis = _dis_from_deg(deg_ref[...])
    pre = jnp.concatenate(
        [agg_ref[0] + hsp_ref[0], agg_ref[1] + hsp_ref[1]], axis=1)
    t = jnp.maximum(dis * pre + b_ref[0], 0.0)
    h = jnp.dot(t, w_ref[...], preferred_element_type=jnp.float32)
    hs = dis * h
    out_ref[0] = hs[:, :HALF]
    out_ref[1] = hs[:, HALF:]


def _tc_out_body(agg_ref, hsp_ref, deg_ref, w_ref, b_ref, bl_ref, out_ref):
    dis = _dis_from_deg(deg_ref[...])
    pre = jnp.concatenate(
        [agg_ref[0] + hsp_ref[0], agg_ref[1] + hsp_ref[1]], axis=1)
    t = jnp.maximum(dis * pre + b_ref[0], 0.0)
    h = jnp.dot(t, w_ref[...], preferred_element_type=jnp.float32)
    out_ref[...] = jax.nn.sigmoid(h + bl_ref[0])


def _stacked_spec():
    return pl.BlockSpec((2, RB, HALF), lambda i: (0, i, 0))


def _tc1(z_p, W0, deg2):
    return pl.pallas_call(
        _tc1_body,
        grid=(NP // RB,),
        in_specs=[
            pl.BlockSpec((RB, IN_C), lambda i: (i, 0)),
            pl.BlockSpec((IN_C, HID), lambda i: (0, 0)),
            pl.BlockSpec((2, RB, DEG_W), lambda i: (0, i, 0)),
        ],
        out_specs=_stacked_spec(),
        out_shape=jax.ShapeDtypeStruct((2, NP, HALF), jnp.float32),
    )(z_p, W0, deg2)


def _tc_mid(agg, hsp, deg2, W1, b0):
    return pl.pallas_call(
        _tc_mid_body,
        grid=(NP // RB,),
        in_specs=[
            _stacked_spec(),
            _stacked_spec(),
            pl.BlockSpec((2, RB, DEG_W), lambda i: (0, i, 0)),
            pl.BlockSpec((HID, HID), lambda i: (0, 0)),
            pl.BlockSpec((1, HID), lambda i: (0, 0)),
        ],
        out_specs=_stacked_spec(),
        out_shape=jax.ShapeDtypeStruct((2, NP, HALF), jnp.float32),
    )(agg, hsp, deg2, W1, b0.reshape(1, HID))


def _tc_out(agg, hsp, deg2, Wl, b1, bl):
    return pl.pallas_call(
        _tc_out_body,
        grid=(NP // RB,),
        in_specs=[
            _stacked_spec(),
            _stacked_spec(),
            pl.BlockSpec((2, RB, DEG_W), lambda i: (0, i, 0)),
            pl.BlockSpec((HID, OUT_C), lambda i: (0, 0)),
            pl.BlockSpec((1, HID), lambda i: (0, 0)),
            pl.BlockSpec((1, OUT_C), lambda i: (0, 0)),
        ],
        out_specs=pl.BlockSpec((RB, OUT_C), lambda i: (i, 0)),
        out_shape=jax.ShapeDtypeStruct((NP, OUT_C), jnp.float32),
    )(agg, hsp, deg2, Wl, b1.reshape(1, HID), bl.reshape(1, OUT_C))


def kernel(z, edge_index, W0, b0, W1, b1, Wl, bl):
    src = edge_index[0]
    dst = edge_index[1]
    pad = (N + (jnp.arange(EP_DEG - E) % (NP - N))).astype(jnp.int32)
    src_p = jnp.concatenate([src, pad])
    dst_p = jnp.concatenate([dst, pad])
    ei_chunks = jnp.stack(
        [src_p[:EP2].reshape(NCHUNK, CHUNK), dst_p[:EP2].reshape(NCHUNK, CHUNK)],
        axis=1)
    z_p = jnp.zeros((NP, IN_C), z.dtype).at[:N].set(z)

    zeros_deg = jnp.zeros((ROWS_PER_TILE, DEG_W), jnp.float32)
    zeros_agg = jnp.zeros((ROWS_PER_TILE, HALF), jnp.float32)

    dst_deg = dst_p.reshape(NCORE * NSUB * DEG_CPT, DEG_CHUNK)
    deg2 = _deg_kernel(dst_deg, zeros_deg)
    hs0 = _tc1(z_p, W0, deg2)
    agg0 = _agg_kernel(hs0, ei_chunks, zeros_agg)
    hs1 = _tc_mid(agg0, hs0, deg2, W1, b0)
    agg1 = _agg_kernel(hs1, ei_chunks, zeros_agg)
    out = _tc_out(agg1, hs1, deg2, Wl, b1, bl)
    return out[:N]


# speedup vs baseline: 20.9649x; 1.3508x over previous
import functools

import jax
import jax.numpy as jnp
from jax import lax
from jax.experimental import pallas as pl
from jax.experimental.pallas import tpu as pltpu
from jax.experimental.pallas import tpu_sc as plsc

N = 10000
NP = 10240
E = 320000
IN_C = 128
HID = 256
OUT_C = 128
HALF = HID // 2

NCORE = 2
NSUB = 16
ROWS_PER_TILE = NP // NSUB

CHUNK = 96
CPT = 211
NCHUNK = NSUB * CPT + 2
EP2 = NCHUNK * CHUNK
DEG_CHUNK = 64
DEG_CPT = 160
EP_DEG = NCORE * NSUB * DEG_CPT * DEG_CHUNK
DEG_W = 16

_mesh = plsc.VectorSubcoreMesh(core_axis_name="c", subcore_axis_name="s")


DEG_DEPTH = 8


@functools.partial(
    pl.kernel,
    out_type=jax.ShapeDtypeStruct((NCORE, NP, DEG_W), jnp.float32),
    mesh=_mesh,
    scratch_types=[
        pltpu.VMEM((DEG_CPT, DEG_CHUNK), jnp.int32),
        pltpu.VMEM((DEG_CHUNK, DEG_W), jnp.float32),
        pltpu.VMEM_SHARED((NP, DEG_W), jnp.float32),
        pltpu.SemaphoreType.DMA,
    ],
)
def _deg_kernel(dst_hbm, zeros_hbm, deg_out, idx_all, ones_v, acc_sh, sem):
    c = lax.axis_index("c")
    s = lax.axis_index("s")
    pltpu.sync_copy(zeros_hbm, acc_sh.at[pl.ds(s * ROWS_PER_TILE, ROWS_PER_TILE)])
    wid = s * NCORE + c
    pltpu.sync_copy(dst_hbm.at[pl.ds(wid * DEG_CPT, DEG_CPT)], idx_all)
    for i in range(DEG_CHUNK):
        ones_v[i, :] = jnp.full((DEG_W,), 1.0, jnp.float32)
    plsc.subcore_barrier()

    def wait_one():
        pltpu.make_async_copy(ones_v, acc_sh.at[idx_all.at[0]], sem).wait()

    def body(k, carry):
        pltpu.async_copy(ones_v, acc_sh.at[idx_all.at[k]], sem, add=True)

        @pl.when(k >= DEG_DEPTH)
        def _():
            wait_one()

        return carry

    lax.fori_loop(0, DEG_CPT, body, 0)
    for _ in range(DEG_DEPTH):
        wait_one()
    plsc.subcore_barrier()
    pltpu.sync_copy(
        acc_sh.at[pl.ds(s * ROWS_PER_TILE, ROWS_PER_TILE)],
        deg_out.at[c, pl.ds(s * ROWS_PER_TILE, ROWS_PER_TILE)],
    )


@functools.partial(
    pl.kernel,
    out_type=jax.ShapeDtypeStruct((NCORE, NP, HALF), jnp.float32),
    mesh=_mesh,
    scratch_types=[
        pltpu.VMEM((2, CHUNK), jnp.int32),
        pltpu.VMEM((2, CHUNK), jnp.int32),
        pltpu.VMEM((2, CHUNK), jnp.int32),
        pltpu.VMEM((CHUNK, HALF), jnp.float32),
        pltpu.VMEM((CHUNK, HALF), jnp.float32),
        pltpu.VMEM((CHUNK, HALF), jnp.float32),
        pltpu.VMEM_SHARED((NP, HALF), jnp.float32),
        pltpu.SemaphoreType.DMA,
        pltpu.SemaphoreType.DMA,
        pltpu.SemaphoreType.DMA,
        pltpu.SemaphoreType.DMA,
        pltpu.SemaphoreType.DMA,
        pltpu.SemaphoreType.DMA,
    ],
)
def _agg_kernel(hs_hbm, ei_hbm, zeros_hbm, out_hbm,
                idx0, idx1, idx2, rows0, rows1, rows2, acc_sh,
                g0, g1, g2, s0, s1, s2):
    c = lax.axis_index("c")
    s = lax.axis_index("s")
    pltpu.sync_copy(zeros_hbm, acc_sh.at[pl.ds(s * ROWS_PER_TILE, ROWS_PER_TILE)])
    plsc.subcore_barrier()
    table = hs_hbm.at[c]
    idx = (idx0, idx1, idx2)
    rows = (rows0, rows1, rows2)
    gsem = (g0, g1, g2)
    ssem = (s0, s1, s2)

    def idx_load(cc, b):
        pltpu.sync_copy(ei_hbm.at[s * CPT + cc], idx[b])

    def gather_start(b):
        pltpu.async_copy(table.at[idx[b].at[0]], rows[b], gsem[b])

    def gather_wait(b):
        pltpu.make_async_copy(table.at[idx[b].at[0]], rows[b], gsem[b]).wait()

    def scat_start(b):
        pltpu.async_copy(rows[b], acc_sh.at[idx[b].at[1]], ssem[b], add=True)

    def scat_wait(b):
        pltpu.make_async_copy(rows[b], acc_sh.at[idx[b].at[1]], ssem[b]).wait()

    def step(cc, b):
        b2 = (b + 2) % 3
        gather_wait(b)
        scat_start(b)
        scat_wait(b2)
        idx_load(cc + 2, b2)
        gather_start(b2)

    idx_load(0, 0)
    gather_start(0)
    idx_load(1, 1)
    gather_start(1)
    gather_wait(0)
    scat_start(0)
    idx_load(2, 2)
    gather_start(2)

    def body(j, carry):
        step(3 * j + 1, 1)
        step(3 * j + 2, 2)
        step(3 * j + 3, 0)
        return carry

    lax.fori_loop(0, (CPT - 1) // 3, body, 0)
    gather_wait(1)
    gather_wait(2)
    scat_wait(0)
    plsc.subcore_barrier()
    pltpu.sync_copy(
        acc_sh.at[pl.ds(s * ROWS_PER_TILE, ROWS_PER_TILE)],
        out_hbm.at[c, pl.ds(s * ROWS_PER_TILE, ROWS_PER_TILE)],
    )


RB = 1024


def _dis_from_deg(deg_blk):
    d = deg_blk[0, :, 0] + deg_blk[1, :, 0] + 1.0
    return (1.0 / jnp.sqrt(d))[:, None]


def _tc1_body(z_ref, w_ref, deg_ref, hs_ref):
    dis = _dis_from_deg(deg_ref[...])
    h = jnp.dot(z_ref[...], w_ref[...], preferred_element_type=jnp.float32)
    hs = dis * h
    hs_ref[0] = hs[:, :HALF]
    hs_ref[1] = hs[:, HALF:]


def _tc_mid_body(agg_ref, hsp_ref, deg_ref, w_ref, b_ref, out_ref):
    dis = _dis_from_deg(deg_ref[...])
    pre = jnp.concatenate(
        [agg_ref[0] + hsp_ref[0], agg_ref[1] + hsp_ref[1]], axis=1)
    t = jnp.maximum(dis * pre + b_ref[0], 0.0)
    h = jnp.dot(t, w_ref[...], preferred_element_type=jnp.float32)
    hs = dis * h
    out_ref[0] = hs[:, :HALF]
    out_ref[1] = hs[:, HALF:]


def _tc_out_body(agg_ref, hsp_ref, deg_ref, w_ref, b_ref, bl_ref, out_ref):
    dis = _dis_from_deg(deg_ref[...])
    pre = jnp.concatenate(
        [agg_ref[0] + hsp_ref[0], agg_ref[1] + hsp_ref[1]], axis=1)
    t = jnp.maximum(dis * pre + b_ref[0], 0.0)
    h = jnp.dot(t, w_ref[...], preferred_element_type=jnp.float32)
    out_ref[...] = jax.nn.sigmoid(h + bl_ref[0])


def _stacked_spec():
    return pl.BlockSpec((2, RB, HALF), lambda i: (0, i, 0))


def _tc1(z_p, W0, deg2):
    return pl.pallas_call(
        _tc1_body,
        grid=(NP // RB,),
        in_specs=[
            pl.BlockSpec((RB, IN_C), lambda i: (i, 0)),
            pl.BlockSpec((IN_C, HID), lambda i: (0, 0)),
            pl.BlockSpec((2, RB, DEG_W), lambda i: (0, i, 0)),
        ],
        out_specs=_stacked_spec(),
        out_shape=jax.ShapeDtypeStruct((2, NP, HALF), jnp.float32),
    )(z_p, W0, deg2)


def _tc_mid(agg, hsp, deg2, W1, b0):
    return pl.pallas_call(
        _tc_mid_body,
        grid=(NP // RB,),
        in_specs=[
            _stacked_spec(),
            _stacked_spec(),
            pl.BlockSpec((2, RB, DEG_W), lambda i: (0, i, 0)),
            pl.BlockSpec((HID, HID), lambda i: (0, 0)),
            pl.BlockSpec((1, HID), lambda i: (0, 0)),
        ],
        out_specs=_stacked_spec(),
        out_shape=jax.ShapeDtypeStruct((2, NP, HALF), jnp.float32),
    )(agg, hsp, deg2, W1, b0.reshape(1, HID))


def _tc_out(agg, hsp, deg2, Wl, b1, bl):
    return pl.pallas_call(
        _tc_out_body,
        grid=(NP // RB,),
        in_specs=[
            _stacked_spec(),
            _stacked_spec(),
            pl.BlockSpec((2, RB, DEG_W), lambda i: (0, i, 0)),
            pl.BlockSpec((HID, OUT_C), lambda i: (0, 0)),
            pl.BlockSpec((1, HID), lambda i: (0, 0)),
            pl.BlockSpec((1, OUT_C), lambda i: (0, 0)),
        ],
        out_specs=pl.BlockSpec((RB, OUT_C), lambda i: (i, 0)),
        out_shape=jax.ShapeDtypeStruct((NP, OUT_C), jnp.float32),
    )(agg, hsp, deg2, Wl, b1.reshape(1, HID), bl.reshape(1, OUT_C))


def kernel(z, edge_index, W0, b0, W1, b1, Wl, bl):
    src = edge_index[0]
    dst = edge_index[1]
    pad = (N + (jnp.arange(EP_DEG - E) % (NP - N))).astype(jnp.int32)
    src_p = jnp.concatenate([src, pad])
    dst_p = jnp.concatenate([dst, pad])
    ei_chunks = jnp.stack(
        [src_p[:EP2].reshape(NCHUNK, CHUNK), dst_p[:EP2].reshape(NCHUNK, CHUNK)],
        axis=1)
    z_p = jnp.zeros((NP, IN_C), z.dtype).at[:N].set(z)

    zeros_deg = jnp.zeros((ROWS_PER_TILE, DEG_W), jnp.float32)
    zeros_agg = jnp.zeros((ROWS_PER_TILE, HALF), jnp.float32)

    dst_deg = dst_p.reshape(NCORE * NSUB * DEG_CPT, DEG_CHUNK)
    deg2 = _deg_kernel(dst_deg, zeros_deg)
    hs0 = _tc1(z_p, W0, deg2)
    agg0 = _agg_kernel(hs0, ei_chunks, zeros_agg)
    hs1 = _tc_mid(agg0, hs0, deg2, W1, b0)
    agg1 = _agg_kernel(hs1, ei_chunks, zeros_agg)
    out = _tc_out(agg1, hs1, deg2, Wl, b1, bl)
    return out[:N]
